# baseline (speedup 1.0000x reference)
import functools

import jax
import jax.numpy as jnp
import numpy as np
from jax.experimental import pallas as pl
from jax.experimental.pallas import tpu as pltpu

_THR = np.float32(0.2)
_EPS = np.float32(1e-9)


def _top2(p):
    c = jax.lax.broadcasted_iota(jnp.int32, p.shape, 1)
    m1 = jnp.max(p, axis=1, keepdims=True)
    i1 = jnp.min(jnp.where(p >= m1, c, p.shape[1]), axis=1, keepdims=True)
    p2 = jnp.where(c == i1, jnp.float32(-1.0), p)
    m2 = jnp.max(p2, axis=1, keepdims=True)
    i2 = jnp.min(jnp.where(p2 >= m2, c, p.shape[1]), axis=1, keepdims=True)
    return m1, i1, m2, i2


def _softmax(l):
    e = jnp.exp(l - jnp.max(l, axis=1, keepdims=True))
    return e / jnp.sum(e, axis=1, keepdims=True)


def _router(eo, ei, x_ref, wg_ref, uo_ref, ui_ref, mean_ref, std_ref,
            xh_ref, w16_ref):
    x = x_ref[...]
    xh = (x - mean_ref[...]) / std_ref[...]
    xh_ref[...] = xh.astype(jnp.bfloat16)
    logits = jnp.dot(xh, wg_ref[...], preferred_element_type=jnp.float32)

    po = _softmax(logits[:, 0:eo])
    g1, o1, g2, o2 = _top2(po)
    den = g1 + g2 + _EPS
    go1 = g1 / den
    go2 = g2 / den
    keep2 = (uo_ref[...] < go2 / _THR).astype(jnp.float32)

    qs = [_softmax(logits[:, eo + ei * e: eo + ei * (e + 1)])
          for e in range(eo)]
    qb = jnp.zeros_like(qs[0])
    qc = jnp.zeros_like(qs[0])
    ub = jnp.zeros_like(g1)
    for e in range(eo):
        qb = qb + jnp.where(o1 == e, qs[e], 0.0)
        qc = qc + jnp.where(o2 == e, qs[e], 0.0)
        ub = ub + jnp.where(o1 == e, ui_ref[:, e:e + 1], 0.0)

    q1, i1, q2, i2 = _top2(qb)
    deni = q1 + q2 + _EPS
    gi1 = q1 / deni
    gi2 = q2 / deni
    keep_i2 = (ub < gi2 / _THR).astype(jnp.float32)

    qm, j1, _, _ = _top2(qc)
    w3 = go2 * (qm / (qm + _EPS)) * keep2

    cp = jax.lax.broadcasted_iota(jnp.int32, (x.shape[0], eo * ei), 1)
    w16_ref[...] = (jnp.where(cp == o1 * ei + i1, go1 * gi1, 0.0)
                    + jnp.where(cp == o1 * ei + i2, keep_i2 * go1 * gi2, 0.0)
                    + jnp.where(cp == o2 * ei + j1, w3, 0.0))


def _body(eo, ei, db, hb2, x_ref, wg_ref, uo_ref, ui_ref, mean_ref, std_ref,
          w1_ref, w2_ref, ostd_ref, omean_ref, out_ref, xh_ref, w16_ref,
          hid_ref, hidb_ref):
    npair = eo * ei
    p = pl.program_id(0)
    j = pl.program_id(1)

    @pl.when((p == 0) & (j == 0))
    def _prologue():
        _router(eo, ei, x_ref, wg_ref, uo_ref, ui_ref, mean_ref, std_ref,
                xh_ref, w16_ref)
        out_ref[...] = jnp.zeros_like(out_ref)

    @pl.when(j < 2)
    def _mm1():
        w1b = w1_ref[0].astype(jnp.bfloat16)
        part = jnp.dot(xh_ref[:, pl.ds(j * db, db)], w1b,
                       preferred_element_type=jnp.float32)

        @pl.when(j == 0)
        def _set():
            hid_ref[...] = part

        @pl.when(j == 1)
        def _add():
            hid_ref[...] += part

    @pl.when(j == 2)
    def _act():
        cp = jax.lax.broadcasted_iota(jnp.int32, w16_ref.shape, 1)
        wcol = jnp.sum(jnp.where(cp == p, w16_ref[...], 0.0), axis=1,
                       keepdims=True)
        hidb_ref[...] = (jnp.maximum(hid_ref[...], 0.0)
                         * wcol).astype(jnp.bfloat16)

    @pl.when(j >= 2)
    def _mm2():
        w2b = w2_ref[0].astype(jnp.bfloat16)
        hslice = hidb_ref[:, pl.ds((j - 2) * hb2, hb2)]
        out_ref[...] += jnp.dot(hslice, w2b,
                                preferred_element_type=jnp.float32)

    @pl.when((p == npair - 1) & (j == 3))
    def _epilogue():
        out_ref[...] = (x_ref[...] + out_ref[...] * ostd_ref[...]
                        + omean_ref[...])


def kernel(x, w_gate_outer, w_gate_inner, w1, w2, input_mean, input_std,
           output_mean, output_std):
    B, D = x.shape
    EO = w_gate_outer.shape[-1]
    EI = w_gate_inner.shape[-1]
    H = w1.shape[-1]
    NP = EO * EI
    DB = D // 2
    HB2 = H // 2

    k1, k2 = jax.random.split(jax.random.key(42))
    u_out = jax.random.uniform(k1, (B, 1), dtype=jnp.float32)
    u_in = jnp.transpose(jax.random.uniform(k2, (EO, B, EI),
                                            dtype=jnp.float32)[:, :, 0])

    wg = jnp.concatenate(
        [w_gate_outer,
         jnp.transpose(w_gate_inner, (1, 0, 2)).reshape(D, EO * EI)], axis=1)

    w1f = w1.reshape(NP, D, H)
    w2f = w2.reshape(NP, H, D)

    out = pl.pallas_call(
        functools.partial(_body, EO, EI, DB, HB2),
        grid=(NP, 4),
        in_specs=[
            pl.BlockSpec((B, D), lambda p, j: (0, 0)),
            pl.BlockSpec(wg.shape, lambda p, j: (0, 0)),
            pl.BlockSpec((B, 1), lambda p, j: (0, 0)),
            pl.BlockSpec((B, EO), lambda p, j: (0, 0)),
            pl.BlockSpec((1, D), lambda p, j: (0, 0)),
            pl.BlockSpec((1, D), lambda p, j: (0, 0)),
            pl.BlockSpec((1, DB, H), lambda p, j: (p, jnp.minimum(j, 1), 0)),
            pl.BlockSpec((1, HB2, D),
                         lambda p, j: (p, jnp.maximum(j - 2, 0), 0)),
            pl.BlockSpec((1, D), lambda p, j: (0, 0)),
            pl.BlockSpec((1, D), lambda p, j: (0, 0)),
        ],
        out_specs=pl.BlockSpec((B, D), lambda p, j: (0, 0)),
        out_shape=jax.ShapeDtypeStruct((B, D), jnp.float32),
        scratch_shapes=[
            pltpu.VMEM((B, D), jnp.bfloat16),
            pltpu.VMEM((B, NP), jnp.float32),
            pltpu.VMEM((B, H), jnp.float32),
            pltpu.VMEM((B, H), jnp.bfloat16),
        ],
        compiler_params=pltpu.CompilerParams(
            dimension_semantics=("arbitrary", "arbitrary")),
    )(x, wg, u_out, u_in, input_mean.reshape(1, D), input_std.reshape(1, D),
      w1f, w2f, output_std.reshape(1, D), output_mean.reshape(1, D))
    return out


# speedup vs baseline: 1.3482x; 1.3482x over previous
import functools

import jax
import jax.numpy as jnp
import numpy as np
from jax.experimental import pallas as pl
from jax.experimental.pallas import tpu as pltpu

_THR = np.float32(0.2)
_EPS = np.float32(1e-9)


def _top2(p):
    c = jax.lax.broadcasted_iota(jnp.int32, p.shape, 1)
    m1 = jnp.max(p, axis=1, keepdims=True)
    i1 = jnp.min(jnp.where(p >= m1, c, p.shape[1]), axis=1, keepdims=True)
    p2 = jnp.where(c == i1, jnp.float32(-1.0), p)
    m2 = jnp.max(p2, axis=1, keepdims=True)
    i2 = jnp.min(jnp.where(p2 >= m2, c, p.shape[1]), axis=1, keepdims=True)
    return m1, i1, m2, i2


def _softmax(l):
    e = jnp.exp(l - jnp.max(l, axis=1, keepdims=True))
    return e / jnp.sum(e, axis=1, keepdims=True)


def _router(eo, ei, x_ref, wg_ref, uo_ref, ui_ref, mean_ref, std_ref,
            xh_ref, w16_ref):
    x = x_ref[...]
    xh = (x - mean_ref[...]) / std_ref[...]
    xh_ref[...] = xh.astype(jnp.bfloat16)
    logits = jnp.dot(xh, wg_ref[...], preferred_element_type=jnp.float32)

    po = _softmax(logits[:, 0:eo])
    g1, o1, g2, o2 = _top2(po)
    den = g1 + g2 + _EPS
    go1 = g1 / den
    go2 = g2 / den
    keep2 = (uo_ref[...] < go2 / _THR).astype(jnp.float32)

    qs = [_softmax(logits[:, eo + ei * e: eo + ei * (e + 1)])
          for e in range(eo)]
    qb = jnp.zeros_like(qs[0])
    qc = jnp.zeros_like(qs[0])
    ub = jnp.zeros_like(g1)
    for e in range(eo):
        qb = qb + jnp.where(o1 == e, qs[e], 0.0)
        qc = qc + jnp.where(o2 == e, qs[e], 0.0)
        ub = ub + jnp.where(o1 == e, ui_ref[:, e:e + 1], 0.0)

    q1, i1, q2, i2 = _top2(qb)
    deni = q1 + q2 + _EPS
    gi1 = q1 / deni
    gi2 = q2 / deni
    keep_i2 = (ub < gi2 / _THR).astype(jnp.float32)

    qm, j1, _, _ = _top2(qc)
    w3 = go2 * (qm / (qm + _EPS)) * keep2

    cp = jax.lax.broadcasted_iota(jnp.int32, (x.shape[0], eo * ei), 1)
    w16_ref[...] = (jnp.where(cp == o1 * ei + i1, go1 * gi1, 0.0)
                    + jnp.where(cp == o1 * ei + i2, keep_i2 * go1 * gi2, 0.0)
                    + jnp.where(cp == o2 * ei + j1, w3, 0.0))


def _body(eo, ei, db, hb2, x_ref, wg_ref, uo_ref, ui_ref, mean_ref, std_ref,
          w1_ref, w2_ref, ostd_ref, omean_ref, out_ref, xh_ref, w16_ref,
          hid0_ref, hid1_ref):
    npair = eo * ei
    p = pl.program_id(0)
    j = pl.program_id(1)

    @pl.when((p == 0) & (j == 0))
    def _prologue():
        _router(eo, ei, x_ref, wg_ref, uo_ref, ui_ref, mean_ref, std_ref,
                xh_ref, w16_ref)
        out_ref[...] = jnp.zeros_like(out_ref)

    for b, hd in ((0, hid0_ref), (1, hid1_ref)):
        @pl.when((p < npair) & ((p & 1) == b))
        def _mm1(hd=hd):
            w1b = w1_ref[0, 0].astype(jnp.bfloat16)
            part = jnp.dot(xh_ref[:, pl.ds(j * db, db)], w1b,
                           preferred_element_type=jnp.float32)

            @pl.when(j == 0)
            def _set():
                hd[...] = part

            @pl.when(j == 1)
            def _add():
                hd[...] += part

    for b, hd in ((0, hid0_ref), (1, hid1_ref)):
        @pl.when((p >= 1) & (((p - 1) & 1) == b))
        def _mm2(hd=hd):
            cp = jax.lax.broadcasted_iota(jnp.int32, w16_ref.shape, 1)
            wcol = jnp.sum(jnp.where(cp == p - 1, w16_ref[...], 0.0),
                           axis=1, keepdims=True)
            hs = jnp.maximum(hd[:, pl.ds(j * hb2, hb2)], 0.0) * wcol
            w2b = w2_ref[0, 0].astype(jnp.bfloat16)
            out_ref[...] += jnp.dot(hs.astype(jnp.bfloat16), w2b,
                                    preferred_element_type=jnp.float32)

    @pl.when((p == npair) & (j == 1))
    def _epilogue():
        out_ref[...] = (x_ref[...] + out_ref[...] * ostd_ref[...]
                        + omean_ref[...])


def kernel(x, w_gate_outer, w_gate_inner, w1, w2, input_mean, input_std,
           output_mean, output_std):
    B, D = x.shape
    EO = w_gate_outer.shape[-1]
    EI = w_gate_inner.shape[-1]
    H = w1.shape[-1]
    NP = EO * EI
    DB = D // 2
    HB2 = H // 2

    k1, k2 = jax.random.split(jax.random.key(42))
    u_out = jax.random.uniform(k1, (B, 1), dtype=jnp.float32)
    u_in = jnp.transpose(jax.random.uniform(k2, (EO, B, EI),
                                            dtype=jnp.float32)[:, :, 0])

    wg = jnp.concatenate(
        [w_gate_outer,
         jnp.transpose(w_gate_inner, (1, 0, 2)).reshape(D, EO * EI)], axis=1)

    w1f = w1.reshape(NP, 2, DB, H)
    w2f = w2.reshape(NP, 2, HB2, D)

    out = pl.pallas_call(
        functools.partial(_body, EO, EI, DB, HB2),
        grid=(NP + 1, 2),
        in_specs=[
            pl.BlockSpec((B, D), lambda p, j: (0, 0)),
            pl.BlockSpec(wg.shape, lambda p, j: (0, 0)),
            pl.BlockSpec((B, 1), lambda p, j: (0, 0)),
            pl.BlockSpec((B, EO), lambda p, j: (0, 0)),
            pl.BlockSpec((1, D), lambda p, j: (0, 0)),
            pl.BlockSpec((1, D), lambda p, j: (0, 0)),
            pl.BlockSpec(
                (1, 1, DB, H),
                lambda p, j: (jnp.minimum(p, NP - 1),
                              jnp.where(p == NP, 1, j), 0, 0)),
            pl.BlockSpec(
                (1, 1, HB2, D),
                lambda p, j: (jnp.maximum(p - 1, 0),
                              jnp.where(p == 0, 0, j), 0, 0)),
            pl.BlockSpec((1, D), lambda p, j: (0, 0)),
            pl.BlockSpec((1, D), lambda p, j: (0, 0)),
        ],
        out_specs=pl.BlockSpec((B, D), lambda p, j: (0, 0)),
        out_shape=jax.ShapeDtypeStruct((B, D), jnp.float32),
        scratch_shapes=[
            pltpu.VMEM((B, D), jnp.bfloat16),
            pltpu.VMEM((B, NP), jnp.float32),
            pltpu.VMEM((B, H), jnp.float32),
            pltpu.VMEM((B, H), jnp.float32),
        ],
        compiler_params=pltpu.CompilerParams(
            dimension_semantics=("arbitrary", "arbitrary"),
            vmem_limit_bytes=80 * 1024 * 1024),
    )(x, wg, u_out, u_in, input_mean.reshape(1, D), input_std.reshape(1, D),
      w1f, w2f, output_std.reshape(1, D), output_mean.reshape(1, D))
    return out


# speedup vs baseline: 1.4592x; 1.0823x over previous
import functools

import jax
import jax.numpy as jnp
from jax.experimental import pallas as pl
from jax.experimental.pallas import tpu as pltpu

import numpy as np

_THR = np.float32(0.2)
_EPS = np.float32(1e-9)


def _top2(p):
    c = jax.lax.broadcasted_iota(jnp.int32, p.shape, 1)
    m1 = jnp.max(p, axis=1, keepdims=True)
    i1 = jnp.min(jnp.where(p >= m1, c, p.shape[1]), axis=1, keepdims=True)
    p2 = jnp.where(c == i1, jnp.float32(-1.0), p)
    m2 = jnp.max(p2, axis=1, keepdims=True)
    i2 = jnp.min(jnp.where(p2 >= m2, c, p.shape[1]), axis=1, keepdims=True)
    return m1, i1, m2, i2


def _softmax(l):
    e = jnp.exp(l - jnp.max(l, axis=1, keepdims=True))
    return e / jnp.sum(e, axis=1, keepdims=True)


def _router_body(eo, ei, x_ref, wg_ref, uo_ref, ui_ref, mean_ref, std_ref,
                 xh_ref, w16_ref):
    x = x_ref[...]
    xh = (x - mean_ref[...]) / std_ref[...]
    xh_ref[...] = xh.astype(jnp.bfloat16)
    logits = jnp.dot(xh, wg_ref[...], preferred_element_type=jnp.float32)

    po = _softmax(logits[:, 0:eo])
    g1, o1, g2, o2 = _top2(po)
    den = g1 + g2 + _EPS
    go1 = g1 / den
    go2 = g2 / den
    keep2 = (uo_ref[...] < go2 / _THR).astype(jnp.float32)

    qs = [_softmax(logits[:, eo + ei * e: eo + ei * (e + 1)]) for e in range(eo)]
    zero = jnp.zeros_like(qs[0])
    qb = zero
    qc = zero
    ub = jnp.zeros_like(g1)
    for e in range(eo):
        qb = qb + jnp.where(o1 == e, qs[e], 0.0)
        qc = qc + jnp.where(o2 == e, qs[e], 0.0)
        ub = ub + jnp.where(o1 == e, ui_ref[:, e:e + 1], 0.0)

    q1, i1, q2, i2 = _top2(qb)
    deni = q1 + q2 + _EPS
    gi1 = q1 / deni
    gi2 = q2 / deni
    keep_i2 = (ub < gi2 / _THR).astype(jnp.float32)

    qm, j1, _, _ = _top2(qc)
    w3 = go2 * (qm / (qm + _EPS)) * keep2

    cp = jax.lax.broadcasted_iota(jnp.int32, (x.shape[0], eo * ei), 1)
    w16 = (jnp.where(cp == o1 * ei + i1, go1 * gi1, 0.0)
           + jnp.where(cp == o1 * ei + i2, keep_i2 * go1 * gi2, 0.0)
           + jnp.where(cp == o2 * ei + j1, w3, 0.0))
    w16_ref[...] = w16


def _ffn_body(eo, ei, np_, nh, x_ref, wg_ref, uo_ref, ui_ref, mean_ref,
              std_ref, w1_ref, w2_ref, ostd_ref, omean_ref, out_ref,
              xh_ref, w16_ref):
    p = pl.program_id(0)
    h = pl.program_id(1)

    @pl.when((p == 0) & (h == 0))
    def _init():
        _router_body(eo, ei, x_ref, wg_ref, uo_ref, ui_ref, mean_ref,
                     std_ref, xh_ref, w16_ref)
        out_ref[...] = jnp.zeros_like(out_ref)

    w1b = w1_ref[0].astype(jnp.bfloat16)
    hid = jnp.dot(xh_ref[...], w1b, preferred_element_type=jnp.float32)
    hid = jnp.maximum(hid, 0.0)
    cp = jax.lax.broadcasted_iota(jnp.int32, w16_ref.shape, 1)
    wcol = jnp.sum(jnp.where(cp == p, w16_ref[...], 0.0), axis=1, keepdims=True)
    hid = (hid * wcol).astype(jnp.bfloat16)
    w2b = w2_ref[0].astype(jnp.bfloat16)
    out_ref[...] += jnp.dot(hid, w2b, preferred_element_type=jnp.float32)

    @pl.when((p == np_ - 1) & (h == nh - 1))
    def _fin():
        out_ref[...] = (x_ref[...] + out_ref[...] * ostd_ref[...]
                        + omean_ref[...])


def kernel(x, w_gate_outer, w_gate_inner, w1, w2, input_mean, input_std,
           output_mean, output_std):
    B, D = x.shape
    EO = w_gate_outer.shape[-1]
    EI = w_gate_inner.shape[-1]
    H = w1.shape[-1]
    NP = EO * EI
    HB = 2048
    NH = H // HB

    k1, k2 = jax.random.split(jax.random.key(42))
    u_out = jax.random.uniform(k1, (B, 1), dtype=jnp.float32)
    u_in = jnp.transpose(jax.random.uniform(k2, (EO, B, EI),
                                            dtype=jnp.float32)[:, :, 0])

    wg = jnp.concatenate(
        [w_gate_outer,
         jnp.transpose(w_gate_inner, (1, 0, 2)).reshape(D, EO * EI)], axis=1)

    w1f = w1.reshape(NP, D, H)
    w2f = w2.reshape(NP, H, D)

    out = pl.pallas_call(
        functools.partial(_ffn_body, EO, EI, NP, NH),
        grid=(NP, NH),
        in_specs=[
            pl.BlockSpec((B, D), lambda p, h: (0, 0)),
            pl.BlockSpec(wg.shape, lambda p, h: (0, 0)),
            pl.BlockSpec((B, 1), lambda p, h: (0, 0)),
            pl.BlockSpec((B, EO), lambda p, h: (0, 0)),
            pl.BlockSpec((1, D), lambda p, h: (0, 0)),
            pl.BlockSpec((1, D), lambda p, h: (0, 0)),
            pl.BlockSpec((1, D, HB), lambda p, h: (p, 0, h)),
            pl.BlockSpec((1, HB, D), lambda p, h: (p, h, 0)),
            pl.BlockSpec((1, D), lambda p, h: (0, 0)),
            pl.BlockSpec((1, D), lambda p, h: (0, 0)),
        ],
        out_specs=pl.BlockSpec((B, D), lambda p, h: (0, 0)),
        out_shape=jax.ShapeDtypeStruct((B, D), jnp.float32),
        scratch_shapes=[
            pltpu.VMEM((B, D), jnp.bfloat16),
            pltpu.VMEM((B, NP), jnp.float32),
        ],
        compiler_params=pltpu.CompilerParams(
            dimension_semantics=("arbitrary", "arbitrary"),
            vmem_limit_bytes=80 * 1024 * 1024),
    )(x, wg, u_out, u_in, input_mean.reshape(1, D), input_std.reshape(1, D),
      w1f, w2f, output_std.reshape(1, D), output_mean.reshape(1, D))
    return out
